# Initial kernel scaffold; baseline (speedup 1.0000x reference)
import jax
import jax.numpy as jnp
from jax.experimental import pallas as pl


def kernel(grid, timestamp, event, train_mode, grid_table, time_table, event_table):
    raise NotImplementedError("write your pallas kernel here")


# speedup vs baseline: 3.3755x; 3.3755x over previous
import functools

import jax
import jax.numpy as jnp
from jax import lax
from jax.experimental import pallas as pl
from jax.experimental.pallas import tpu as pltpu
from jax.experimental.pallas import tpu_sc as plsc

B, L, D = 4096, 200, 16
N = B * L
NW = 32
TOK_PER_W = N // NW
G = 128
NG = 4
C = G * NG
NCH = TOK_PER_W // C
NROW = N // G

_mesh = plsc.VectorSubcoreMesh(core_axis_name="c", subcore_axis_name="s")


@functools.partial(
    pl.kernel,
    out_type=jax.ShapeDtypeStruct((NROW, G, D), jnp.float32),
    mesh=_mesh,
    compiler_params=pltpu.CompilerParams(use_tc_tiling_on_sc=False),
    scratch_types=[
        pltpu.VMEM((NG, G), jnp.int32),
        pltpu.VMEM((NG, G), jnp.int32),
        pltpu.VMEM((NG, G), jnp.int32),
        pltpu.VMEM((NG, G, D), jnp.float32),
        pltpu.VMEM((NG, G, D), jnp.float32),
        pltpu.VMEM((NG, G, D), jnp.float32),
        pltpu.SemaphoreType.DMA,
    ],
)
def _embed_sum(gid_hbm, tid_hbm, eid_hbm, gt_hbm, tt_hbm, et_hbm, out_hbm,
               gidx, tidx, eidx, grows, trows, erows, sem):
    wid = lax.axis_index("s") * 2 + lax.axis_index("c")
    wrow0 = wid * (TOK_PER_W // G)

    def chunk_body(ch, carry):
        row0 = wrow0 + ch * NG
        pltpu.sync_copy(gid_hbm.at[pl.ds(row0, NG)], gidx)
        pltpu.sync_copy(tid_hbm.at[pl.ds(row0, NG)], tidx)
        pltpu.sync_copy(eid_hbm.at[pl.ds(row0, NG)], eidx)
        copies = []
        for g in range(NG):
            copies.append(pltpu.async_copy(gt_hbm.at[gidx.at[g]], grows.at[g], sem))
            copies.append(pltpu.async_copy(tt_hbm.at[tidx.at[g]], trows.at[g], sem))
            copies.append(pltpu.async_copy(et_hbm.at[eidx.at[g]], erows.at[g], sem))
        for cp in copies:
            cp.wait()

        for g in range(NG):
            def add_body(i, _):
                grows[g, i] = grows[g, i] + trows[g, i] + erows[g, i]
                return 0
            lax.fori_loop(0, G, add_body, 0, unroll=8)

        pltpu.sync_copy(grows, out_hbm.at[pl.ds(row0, NG)])
        return carry

    lax.fori_loop(0, NCH, chunk_body, 0)


def kernel(grid, timestamp, event, train_mode, grid_table, time_table, event_table):
    gid = grid.reshape(NROW, G).astype(jnp.int32)
    tid = timestamp.reshape(NROW, G).astype(jnp.int32)
    eid = event.reshape(NROW, G).astype(jnp.int32)
    out = _embed_sum(gid, tid, eid, grid_table, time_table, event_table)
    return out.reshape(B, L, D)


# speedup vs baseline: 3.3892x; 1.0040x over previous
import functools

import jax
import jax.numpy as jnp
from jax import lax
from jax.experimental import pallas as pl
from jax.experimental.pallas import tpu as pltpu
from jax.experimental.pallas import tpu_sc as plsc

B, L, D = 4096, 200, 16
N = B * L
NW = 32
TOK_PER_W = N // NW
G = 128
NG = 4
C = G * NG
NCH = TOK_PER_W // C
NROW = N // G
NBUF = 2

_mesh = plsc.VectorSubcoreMesh(core_axis_name="c", subcore_axis_name="s")


@functools.partial(
    pl.kernel,
    out_type=jax.ShapeDtypeStruct((NROW, G, D), jnp.float32),
    mesh=_mesh,
    compiler_params=pltpu.CompilerParams(use_tc_tiling_on_sc=False),
    scratch_types=(
        [pltpu.VMEM((NG, G), jnp.int32) for _ in range(3 * NBUF)]
        + [pltpu.VMEM((NG, G, D), jnp.float32) for _ in range(4 * NBUF)]
        + [pltpu.SemaphoreType.DMA for _ in range(2 * NBUF)]
    ),
)
def _embed_sum(gid_hbm, tid_hbm, eid_hbm, gt_hbm, tt_hbm, et_hbm, out_hbm,
               *scr):
    idxs = [scr[0:3], scr[3:6]]
    rows = [scr[6:9], scr[9:12]]
    obuf = scr[12:14]
    gsem = scr[14:16]
    osem = scr[16:18]

    wid = lax.axis_index("s") * 2 + lax.axis_index("c")
    wrow0 = wid * (TOK_PER_W // G)
    tabs = (gt_hbm, tt_hbm, et_hbm)
    ids = (gid_hbm, tid_hbm, eid_hbm)

    def stage(ch, b):
        row0 = wrow0 + ch * NG
        for t in range(3):
            pltpu.sync_copy(ids[t].at[pl.ds(row0, NG)], idxs[b][t])
        for t in range(3):
            for g in range(NG):
                pltpu.async_copy(tabs[t].at[idxs[b][t].at[g]],
                                 rows[b][t].at[g], gsem[b])

    def wait_gathers(b):
        for t in range(3):
            for g in range(NG):
                pltpu.make_async_copy(tabs[t].at[idxs[b][t].at[g]],
                                      rows[b][t].at[g], gsem[b]).wait()

    def wait_out(b):
        pltpu.make_async_copy(obuf[b], out_hbm.at[pl.ds(0, NG)],
                              osem[b]).wait()

    stage(0, 0)
    stage(1, 1)

    def pair_body(k, carry):
        for b in range(NBUF):
            ch = k * NBUF + b
            wait_gathers(b)

            @pl.when(ch >= NBUF)
            def _():
                wait_out(b)

            gr, tr, er = rows[b]
            ob = obuf[b]
            for g in range(NG):
                @plsc.parallel_loop(0, G, unroll=8)
                def _(j):
                    ob[g, j] = gr[g, j] + tr[g, j] + er[g, j]

            @pl.when(ch + NBUF < NCH)
            def _():
                stage(ch + NBUF, b)

            pltpu.async_copy(ob, out_hbm.at[pl.ds(wrow0 + ch * NG, NG)],
                             osem[b])
        return carry

    lax.fori_loop(0, NCH // NBUF, pair_body, 0)
    for b in range(NBUF):
        wait_out(b)


def kernel(grid, timestamp, event, train_mode, grid_table, time_table, event_table):
    gid = grid.reshape(NROW, G).astype(jnp.int32)
    tid = timestamp.reshape(NROW, G).astype(jnp.int32)
    eid = event.reshape(NROW, G).astype(jnp.int32)
    out = _embed_sum(gid, tid, eid, grid_table, time_table, event_table)
    return out.reshape(B, L, D)


# speedup vs baseline: 4.6623x; 1.3757x over previous
import functools

import jax
import jax.numpy as jnp
from jax import lax
from jax.experimental import pallas as pl
from jax.experimental.pallas import tpu as pltpu
from jax.experimental.pallas import tpu_sc as plsc

B, L, D = 4096, 200, 16
N = B * L
NW = 32
TOK_PER_W = N // NW
G = 128
NG = 2
C = G * NG
NCH = TOK_PER_W // C
NBUF = 4
TV, EV = 52, 103

_mesh = plsc.VectorSubcoreMesh(core_axis_name="c", subcore_axis_name="s")


@functools.partial(
    pl.kernel,
    out_type=jax.ShapeDtypeStruct((N, D), jnp.float32),
    mesh=_mesh,
    compiler_params=pltpu.CompilerParams(use_tc_tiling_on_sc=False,
                                         needs_layout_passes=False),
    scratch_types=(
        [pltpu.VMEM((C,), jnp.int32) for _ in range(3 * NBUF)]
        + [pltpu.VMEM((C, D), jnp.float32) for _ in range(NBUF)]
        + [pltpu.VMEM((TV * D,), jnp.float32),
           pltpu.VMEM((EV * D,), jnp.float32)]
        + [pltpu.SemaphoreType.DMA for _ in range(3 * NBUF)]
    ),
)
def _embed_sum(gid_hbm, tid_hbm, eid_hbm, gt_hbm, tt_hbm, et_hbm, out_hbm,
               *scr):
    gidx = scr[0:4]
    tidx = scr[4:8]
    eidx = scr[8:12]
    ob = scr[12:16]
    ttv, etv = scr[16], scr[17]
    isem = scr[18:22]
    gsem = scr[22:26]
    osem = scr[26:30]

    wid = lax.axis_index("s") * 2 + lax.axis_index("c")
    wtok0 = wid * TOK_PER_W

    pltpu.sync_copy(tt_hbm, ttv)
    pltpu.sync_copy(et_hbm, etv)

    def stage_idx(ch, b):
        base = wtok0 + ch * C
        pltpu.async_copy(gid_hbm.at[pl.ds(base, C)], gidx[b], isem[b])
        pltpu.async_copy(tid_hbm.at[pl.ds(base, C)], tidx[b], isem[b])
        pltpu.async_copy(eid_hbm.at[pl.ds(base, C)], eidx[b], isem[b])

    def wait_idx(b):
        for ref in (gidx[b], tidx[b], eidx[b]):
            pltpu.make_async_copy(gid_hbm.at[pl.ds(0, C)], ref,
                                  isem[b]).wait()

    def fire_gathers(b):
        for g in range(NG):
            pltpu.async_copy(gt_hbm.at[gidx[b].at[pl.ds(g * G, G)]],
                             ob[b].at[pl.ds(g * G, G)], gsem[b])

    def wait_gathers(b):
        for g in range(NG):
            pltpu.make_async_copy(gt_hbm.at[gidx[b].at[pl.ds(g * G, G)]],
                                  ob[b].at[pl.ds(g * G, G)], gsem[b]).wait()

    def wait_out(b):
        pltpu.make_async_copy(ob[b], out_hbm.at[pl.ds(0, C)], osem[b]).wait()

    lanes = lax.iota(jnp.int32, 16)
    for c in range(NBUF):
        stage_idx(c, c)
    for c in range(2):
        wait_idx(c)
        fire_gathers(c)

    def pair_body(k, carry):
        for b in range(NBUF):
            ch = k * NBUF + b
            wait_gathers(b)
            o = ob[b]
            ti, ei = tidx[b], eidx[b]

            @plsc.parallel_loop(0, C // 16, unroll=2)
            def _(grp):
                tk = grp * 16
                tvec = ti[pl.ds(tk, 16)] * D
                evec = ei[pl.ds(tk, 16)] * D
                tokv = tk + lanes
                for d in range(D):
                    tt = plsc.load_gather(ttv, [tvec + d])
                    ee = plsc.load_gather(etv, [evec + d])
                    plsc.addupdate_scatter(
                        o, [tokv, jnp.full((16,), d, jnp.int32)], tt + ee)

            pltpu.async_copy(o, out_hbm.at[pl.ds(wtok0 + ch * C, C)], osem[b])

            @pl.when(ch + NBUF < NCH)
            def _():
                stage_idx(ch + NBUF, b)

            bg = (b + 2) % NBUF

            @pl.when(ch + 2 < NCH)
            def _():
                @pl.when(ch >= 2)
                def _():
                    wait_out(bg)
                wait_idx(bg)
                fire_gathers(bg)
        return carry

    lax.fori_loop(0, NCH // NBUF, pair_body, 0)
    for b in range(NBUF):
        wait_out(b)


def kernel(grid, timestamp, event, train_mode, grid_table, time_table, event_table):
    gid = grid.reshape(N).astype(jnp.int32)
    tid = timestamp.reshape(N).astype(jnp.int32)
    eid = event.reshape(N).astype(jnp.int32)
    out = _embed_sum(gid, tid, eid, grid_table,
                     time_table.reshape(TV * D), event_table.reshape(EV * D))
    return out.reshape(B, L, D)
